# Initial kernel scaffold; baseline (speedup 1.0000x reference)
import jax
import jax.numpy as jnp
from jax.experimental import pallas as pl


def kernel(adj, seq1, W, bias, prelu_a):
    raise NotImplementedError("write your pallas kernel here")


# speedup vs baseline: 1.0222x; 1.0222x over previous
import functools

import jax
import jax.numpy as jnp
from jax.experimental import pallas as pl

N = 64
N_IN = 128
N_H = 128
BB = 32


def _fused_gcn_kernel(adj_ref, seq_ref, w_ref, bias_ref, a_ref, out_ref):
    bb = adj_ref.shape[0]
    seq = seq_ref[...].reshape(bb * N, N_IN)
    sf = jnp.dot(seq, w_ref[...], preferred_element_type=jnp.float32)
    sf = sf.reshape(bb, N, N_H)
    out = jax.lax.dot_general(
        adj_ref[...], sf,
        dimension_numbers=(((2,), (1,)), ((0,), (0,))),
        preferred_element_type=jnp.float32,
    )
    row = jax.lax.broadcasted_iota(jnp.int32, (bb, N, N_H), 1)
    out = jnp.where(row < N - 1, out, jnp.inf)
    m = jnp.min(out, axis=1) + bias_ref[...]
    a = a_ref[0, 0]
    out_ref[...] = jnp.where(m >= 0, m, a * m)


@jax.jit
def kernel(adj, seq1, W, bias, prelu_a):
    B = adj.shape[0]
    grid = (B // BB,)
    return pl.pallas_call(
        _fused_gcn_kernel,
        grid=grid,
        in_specs=[
            pl.BlockSpec((BB, N, N), lambda i: (i, 0, 0)),
            pl.BlockSpec((BB, N, N_IN), lambda i: (i, 0, 0)),
            pl.BlockSpec((N_IN, N_H), lambda i: (0, 0)),
            pl.BlockSpec((1, N_H), lambda i: (0, 0)),
            pl.BlockSpec((1, 1), lambda i: (0, 0)),
        ],
        out_specs=pl.BlockSpec((BB, N_H), lambda i: (i, 0)),
        out_shape=jax.ShapeDtypeStruct((B, N_H), jnp.float32),
    )(adj, seq1, W, bias.reshape(1, N_H), prelu_a.reshape(1, 1))


# speedup vs baseline: 1.3275x; 1.2987x over previous
import functools

import jax
import jax.numpy as jnp
from jax.experimental import pallas as pl

N = 64
N_IN = 128
N_H = 128
BB = 128


def _fused_gcn_kernel(adj_ref, seq_ref, w_ref, bias_ref, a_ref, out_ref):
    bb = adj_ref.shape[0]
    seq = seq_ref[...].reshape(bb * N, N_IN)
    sf = jnp.dot(seq, w_ref[...], preferred_element_type=jnp.float32)
    sf = sf.reshape(bb, N, N_H)
    out = jax.lax.dot_general(
        adj_ref[...], sf,
        dimension_numbers=(((2,), (1,)), ((0,), (0,))),
        preferred_element_type=jnp.float32,
    )
    row = jax.lax.broadcasted_iota(jnp.int32, (bb, N, N_H), 1)
    out = jnp.where(row < N - 1, out, jnp.inf)
    m = jnp.min(out, axis=1) + bias_ref[...]
    a = a_ref[0, 0]
    out_ref[...] = jnp.where(m >= 0, m, a * m)


@jax.jit
def kernel(adj, seq1, W, bias, prelu_a):
    B = adj.shape[0]
    grid = (B // BB,)
    return pl.pallas_call(
        _fused_gcn_kernel,
        grid=grid,
        in_specs=[
            pl.BlockSpec((BB, N, N), lambda i: (i, 0, 0)),
            pl.BlockSpec((BB, N, N_IN), lambda i: (i, 0, 0)),
            pl.BlockSpec((N_IN, N_H), lambda i: (0, 0)),
            pl.BlockSpec((1, N_H), lambda i: (0, 0)),
            pl.BlockSpec((1, 1), lambda i: (0, 0)),
        ],
        out_specs=pl.BlockSpec((BB, N_H), lambda i: (i, 0)),
        out_shape=jax.ShapeDtypeStruct((B, N_H), jnp.float32),
    )(adj, seq1, W, bias.reshape(1, N_H), prelu_a.reshape(1, 1))


# speedup vs baseline: 1.8813x; 1.4172x over previous
import jax
import jax.numpy as jnp
from jax.experimental import pallas as pl

N = 64
N_IN = 128
N_H = 128
BB = 128


def _fused_gcn_kernel(adj_ref, seq_ref, w_ref, bias_ref, a_ref, out_ref):
    bb = adj_ref.shape[0]
    seq = seq_ref[...].reshape(bb * N, N_IN)
    sf = jnp.dot(seq, w_ref[...], preferred_element_type=jnp.float32)
    sf = sf.reshape(bb, N, N_H)
    adj2 = adj_ref[...]
    a_even = adj2[:, :, :N]
    a_odd = adj2[:, :, N:]
    dn = (((2,), (1,)), ((0,), (0,)))
    out_e = jax.lax.dot_general(a_even, sf, dn, preferred_element_type=jnp.float32)
    out_o = jax.lax.dot_general(a_odd, sf, dn, preferred_element_type=jnp.float32)
    row = jax.lax.broadcasted_iota(jnp.int32, (bb, N // 2, N_H), 1)
    out_o = jnp.where(row < N // 2 - 1, out_o, jnp.inf)
    m = jnp.minimum(jnp.min(out_e, axis=1), jnp.min(out_o, axis=1))
    m = m + bias_ref[...]
    a = a_ref[0, 0]
    out_ref[...] = jnp.where(m >= 0, m, a * m)


def kernel(adj, seq1, W, bias, prelu_a):
    B = adj.shape[0]
    grid = (B // BB,)
    return pl.pallas_call(
        _fused_gcn_kernel,
        grid=grid,
        in_specs=[
            pl.BlockSpec((BB, N // 2, 2 * N), lambda i: (i, 0, 0)),
            pl.BlockSpec((BB, N, N_IN), lambda i: (i, 0, 0)),
            pl.BlockSpec((N_IN, N_H), lambda i: (0, 0)),
            pl.BlockSpec((1, N_H), lambda i: (0, 0)),
            pl.BlockSpec((1, 1), lambda i: (0, 0)),
        ],
        out_specs=pl.BlockSpec((BB, N_H), lambda i: (i, 0)),
        out_shape=jax.ShapeDtypeStruct((B, N_H), jnp.float32),
    )(adj.reshape(B, N // 2, 2 * N), seq1, W,
      bias.reshape(1, N_H), prelu_a.reshape(1, 1))
